# baseline (speedup 1.0000x reference)
import functools

import jax
import jax.numpy as jnp
from jax import lax
from jax.experimental import pallas as pl
from jax.experimental.pallas import tpu as pltpu
from jax.experimental.pallas import tpu_sc as plsc

import sys
try:
    _probe = jax.device_put(jnp.zeros((1024, 32), jnp.float32))
    print("LAYOUT_PROBE (1024,32):", _probe.format, file=sys.stderr)
    _probe2 = jax.device_put(jnp.zeros((1024, 128), jnp.float32))
    print("LAYOUT_PROBE (1024,128):", _probe2.format, file=sys.stderr)
    _probe3 = jax.device_put(jnp.zeros((16384, 3), jnp.int32))
    print("LAYOUT_PROBE (16384,3):", _probe3.format, file=sys.stderr)
except Exception as _e:
    print("LAYOUT_PROBE failed:", _e, file=sys.stderr)

B = 16384
K = 32
NC = 2
NS = 16
NW = NC * NS
BPW = B // NW
CHUNK = 128
NCHUNK = BPW // CHUNK
L = 16


def _body(s_hbm, p_hbm, o_hbm, ent_hbm, rel_hbm, es_hbm, ep_hbm, eo_hbm,
          idx_s, idx_p, idx_o, rows_s, rows_p, rows_o,
          sem_s, sem_p, sem_o):
    wid = lax.axis_index("c") * NS + lax.axis_index("s")
    base = wid * BPW

    for j in range(NCHUNK):
        sl = pl.ds(base + j * CHUNK, CHUNK)
        pltpu.sync_copy(s_hbm.at[sl], idx_s.at[j])
        pltpu.sync_copy(p_hbm.at[sl], idx_p.at[j])
        pltpu.sync_copy(o_hbm.at[sl], idx_o.at[j])

    copies = []
    for j in range(NCHUNK):
        dst = pl.ds(j * CHUNK, CHUNK)
        copies.append(pltpu.async_copy(ent_hbm.at[idx_s.at[j]], rows_s.at[dst], sem_s))
        copies.append(pltpu.async_copy(rel_hbm.at[idx_p.at[j]], rows_p.at[dst], sem_p))
        copies.append(pltpu.async_copy(ent_hbm.at[idx_o.at[j]], rows_o.at[dst], sem_o))
    for c in copies:
        c.wait()

    out_sl = pl.ds(base, BPW)
    pltpu.sync_copy(rows_s, es_hbm.at[out_sl])
    pltpu.sync_copy(rows_p, ep_hbm.at[out_sl])
    pltpu.sync_copy(rows_o, eo_hbm.at[out_sl])


@jax.jit
def kernel(triples, ent_emb, rel_emb):
    out = jax.ShapeDtypeStruct((B, K), jnp.float32)
    mesh = plsc.VectorSubcoreMesh(core_axis_name="c", subcore_axis_name="s")
    f = pl.kernel(
        _body,
        out_type=(out, out, out),
        mesh=mesh,
        compiler_params=pltpu.CompilerParams(use_tc_tiling_on_sc=False),
        scratch_types=[
            pltpu.VMEM((NCHUNK, CHUNK), jnp.int32),
            pltpu.VMEM((NCHUNK, CHUNK), jnp.int32),
            pltpu.VMEM((NCHUNK, CHUNK), jnp.int32),
            pltpu.VMEM((BPW, K), jnp.float32),
            pltpu.VMEM((BPW, K), jnp.float32),
            pltpu.VMEM((BPW, K), jnp.float32),
            pltpu.SemaphoreType.DMA,
            pltpu.SemaphoreType.DMA,
            pltpu.SemaphoreType.DMA,
        ],
    )
    one = jnp.float32(1) + jnp.float32(0) * jax.lax.optimization_barrier(ent_emb[0, 0])
    return f(triples[:, 0], triples[:, 1], triples[:, 2], ent_emb * one, rel_emb)


# speedup vs baseline: 1.5939x; 1.5939x over previous
import jax
import jax.numpy as jnp
from jax import lax
from jax.experimental import pallas as pl
from jax.experimental.pallas import tpu as pltpu
from jax.experimental.pallas import tpu_sc as plsc

B = 16384
K = 32
NC = 2
NS = 16
NW = NC * NS
BPW = B // NW
CHUNK = 128
NCHUNK = BPW // CHUNK


def _body(s_hbm, p_hbm, o_hbm, ent_hbm, rel_hbm, es_hbm, ep_hbm, eo_hbm,
          idx_s, idx_p, idx_o, rows_s, rows_p, rows_o,
          sem_i, sem_s, sem_p, sem_o):
    wid = lax.axis_index("c") * NS + lax.axis_index("s")
    base = wid * BPW

    stages = []
    for j in range(NCHUNK):
        sl = pl.ds(base + j * CHUNK, CHUNK)
        stages.append((pltpu.async_copy(s_hbm.at[sl], idx_s.at[j], sem_i),
                       pltpu.async_copy(p_hbm.at[sl], idx_p.at[j], sem_i),
                       pltpu.async_copy(o_hbm.at[sl], idx_o.at[j], sem_i)))

    copies = []
    for j in range(NCHUNK):
        dst = pl.ds(j * CHUNK, CHUNK)
        for st in stages[j]:
            st.wait()
        copies.append(pltpu.async_copy(ent_hbm.at[idx_s.at[j]], rows_s.at[dst], sem_s))
        copies.append(pltpu.async_copy(rel_hbm.at[idx_p.at[j]], rows_p.at[dst], sem_p))
        copies.append(pltpu.async_copy(ent_hbm.at[idx_o.at[j]], rows_o.at[dst], sem_o))
    for c in copies:
        c.wait()

    out_sl = pl.ds(base, BPW)
    outs = [pltpu.async_copy(rows_s, es_hbm.at[out_sl], sem_s),
            pltpu.async_copy(rows_p, ep_hbm.at[out_sl], sem_p),
            pltpu.async_copy(rows_o, eo_hbm.at[out_sl], sem_o)]
    for c in outs:
        c.wait()


@jax.jit
def kernel(triples, ent_emb, rel_emb):
    out = jax.ShapeDtypeStruct((B, K), jnp.float32)
    mesh = plsc.VectorSubcoreMesh(core_axis_name="c", subcore_axis_name="s")
    f = pl.kernel(
        _body,
        out_type=(out, out, out),
        mesh=mesh,
        compiler_params=pltpu.CompilerParams(use_tc_tiling_on_sc=False),
        scratch_types=[
            pltpu.VMEM((NCHUNK, CHUNK), jnp.int32),
            pltpu.VMEM((NCHUNK, CHUNK), jnp.int32),
            pltpu.VMEM((NCHUNK, CHUNK), jnp.int32),
            pltpu.VMEM((BPW, K), jnp.float32),
            pltpu.VMEM((BPW, K), jnp.float32),
            pltpu.VMEM((BPW, K), jnp.float32),
            pltpu.SemaphoreType.DMA,
            pltpu.SemaphoreType.DMA,
            pltpu.SemaphoreType.DMA,
            pltpu.SemaphoreType.DMA,
        ],
    )
    return f(triples[:, 0], triples[:, 1], triples[:, 2], ent_emb, rel_emb)
